# baseline (speedup 1.0000x reference)
import functools

import jax
import jax.numpy as jnp
from jax import lax
from jax.experimental import pallas as pl
from jax.experimental.pallas import tpu as pltpu
from jax.experimental.pallas import tpu_sc as plsc

N = 10000
E = 320000
F = 128
G = 64
NC = 2
NS = 16
NW = NC * NS
K = 80
CHD = (E // NW) // K
CHA = (E // NS) // K
NP = 10240
RPT = NP // NS
HN = NP // 2
ACC_R = HN + 8
RPA = HN // NS
ZR = 64
DW = 16

_mesh = plsc.VectorSubcoreMesh(
    core_axis_name="c", subcore_axis_name="s", num_cores=NC, num_subcores=NS)

_f32 = jnp.float32


@functools.partial(
    pl.kernel,
    out_type=jax.ShapeDtypeStruct((NC, ACC_R, F), _f32),
    mesh=_mesh,
    scratch_types=[
        pltpu.VMEM((CHA, K), jnp.int32),
        pltpu.VMEM((CHA, K), jnp.int32),
        pltpu.VMEM((K, F), _f32),
        pltpu.VMEM((K, F), _f32),
        pltpu.VMEM_SHARED((ACC_R, F), _f32),
        pltpu.SemaphoreType.DMA,
        pltpu.SemaphoreType.DMA,
        pltpu.SemaphoreType.DMA,
        pltpu.SemaphoreType.DMA,
    ],
)
def _agg_kernel(src_hbm, rdst_hbm, hp_hbm, zeros_hbm, out_hbm,
                srcv, dstv, rows0, rows1, accum, gsem0, gsem1, ssem0, ssem1):
    c = lax.axis_index("c")
    s = lax.axis_index("s")

    @pl.when(s == 0)
    def _():
        pltpu.sync_copy(zeros_hbm, accum)
    plsc.subcore_barrier()

    pltpu.sync_copy(src_hbm.at[s], srcv)
    pltpu.sync_copy(rdst_hbm.at[c, s], dstv)

    def scat(rows, j, ssem):
        cps = []
        for q in range(K // 16):
            idx16 = dstv[j, pl.ds(q * 16, 16)]
            cps.append(pltpu.async_copy(rows.at[pl.ds(q * 16, 16)],
                                        accum.at[idx16], ssem, add=True))
        for cp in cps:
            cp.wait()

    pltpu.async_copy(hp_hbm.at[srcv.at[0]], rows0, gsem0)

    def pair(t, _):
        j0 = 2 * t
        j1 = 2 * t + 1
        pltpu.async_copy(hp_hbm.at[srcv.at[j1]], rows1, gsem1)
        pltpu.make_async_copy(hp_hbm.at[srcv.at[j0]], rows0, gsem0).wait()
        scat(rows0, j0, ssem0)

        @pl.when(j0 + 2 < CHA)
        def _():
            pltpu.async_copy(hp_hbm.at[srcv.at[j0 + 2]], rows0, gsem0)
        pltpu.make_async_copy(hp_hbm.at[srcv.at[j1]], rows1, gsem1).wait()
        scat(rows1, j1, ssem1)
        return 0
    lax.fori_loop(0, CHA // 2, pair, 0)

    plsc.subcore_barrier()

    @pl.when(s == 0)
    def _():
        pltpu.sync_copy(accum, out_hbm.at[c])


def _stitch(agg_ref):
    return jnp.concatenate([agg_ref[0, :HN], agg_ref[1, :N - HN]], axis=0)


def _tc_prep(dst_ref, rdst_ref):
    d = dst_ref[...]
    rdst_ref[0] = jnp.where(d < HN, d, HN)
    d1 = d - HN
    rdst_ref[1] = jnp.where(d1 >= 0, d1, HN)


_tc_prep_call = pl.pallas_call(
    _tc_prep,
    out_shape=jax.ShapeDtypeStruct((NC, NS, CHA, K), jnp.int32))


def _tc_a(x_ref, w1t_ref, degagg_ref, hp_ref, dinv_ref):
    deg = 1.0 + _stitch(degagg_ref)[:, 0:1]
    dinv = lax.rsqrt(deg)
    dinv_ref[...] = dinv
    hp_ref[...] = dinv * jnp.dot(x_ref[...], w1t_ref[...],
                                 preferred_element_type=_f32)


_tc_a_call = pl.pallas_call(
    _tc_a,
    out_shape=(jax.ShapeDtypeStruct((N, F), _f32),
               jax.ShapeDtypeStruct((N, 1), _f32)),
)


def _bn_relu(agg, hp, dinv, b, g, be):
    t = dinv * (agg + hp) + b
    mu = jnp.mean(t, axis=0, keepdims=True)
    var = jnp.mean((t - mu) ** 2, axis=0, keepdims=True)
    return jnp.maximum(g * (t - mu) * lax.rsqrt(var + 1e-5) + be, 0.0)


def _tc_b(agg_ref, hp_ref, dinv_ref, b_ref, g_ref, be_ref, w2t_ref, out_ref):
    dinv = dinv_ref[...]
    y = _bn_relu(_stitch(agg_ref), hp_ref[...], dinv,
                 b_ref[...], g_ref[...], be_ref[...])
    out_ref[...] = dinv * jnp.dot(y, w2t_ref[...], preferred_element_type=_f32)


_tc_b_call = pl.pallas_call(
    _tc_b, out_shape=jax.ShapeDtypeStruct((N, F), _f32))


def _tc_c(agg_ref, hp_ref, dinv_ref, b_ref, g_ref, be_ref,
          batch_ref, wl1t_ref, bl1_ref, wl2t_ref, bl2_ref, out_ref):
    dinv = dinv_ref[...]
    y = _bn_relu(_stitch(agg_ref), hp_ref[...], dinv,
                 b_ref[...], g_ref[...], be_ref[...])
    gid = lax.broadcasted_iota(jnp.int32, (G, N), 0)
    oh = (gid == batch_ref[...]).astype(_f32)
    cnt = jnp.maximum(jnp.sum(oh, axis=1, keepdims=True), 1.0)
    pooled = jnp.dot(oh, y, preferred_element_type=_f32) / cnt
    r = jnp.maximum(
        jnp.dot(pooled, wl1t_ref[...], preferred_element_type=_f32)
        + bl1_ref[...], 0.0)
    out_ref[...] = (jnp.dot(r, wl2t_ref[...], preferred_element_type=_f32)
                    + bl2_ref[...])


_tc_c_call = pl.pallas_call(
    _tc_c, out_shape=jax.ShapeDtypeStruct((G, 1), _f32))


def kernel(x, edge_index, batch, W1, b1, g1, be1, W2, b2, g2, be2,
           Wl1, bl1, Wl2, bl2):
    src = edge_index[0].reshape(NS, CHA, K)
    dst = edge_index[1].reshape(NS, CHA, K)
    rdst = _tc_prep_call(dst)
    az = jnp.zeros((ACC_R, F), _f32)
    degagg = _agg_kernel(src, rdst, jnp.ones((N, F), _f32), az)
    hp1, dinv = _tc_a_call(x, W1.T, degagg)
    agg1 = _agg_kernel(src, rdst, hp1, az)
    hp2 = _tc_b_call(agg1, hp1, dinv, b1.reshape(1, F), g1.reshape(1, F),
                     be1.reshape(1, F), W2.T)
    agg2 = _agg_kernel(src, rdst, hp2, az)
    out = _tc_c_call(agg2, hp2, dinv, b2.reshape(1, F), g2.reshape(1, F),
                     be2.reshape(1, F), batch.reshape(1, N), Wl1.T,
                     bl1.reshape(1, G), Wl2.T, bl2.reshape(1, 1))
    return out.reshape(G)
